# baseline (speedup 1.0000x reference)
import jax
import jax.numpy as jnp
from jax import lax
from jax.experimental import pallas as pl
from jax.experimental.pallas import tpu as pltpu
from jax.experimental.pallas import tpu_sc as plsc

VOCAB = 1000000
MAXLEN = 200
EMBED_DIM = 32
BATCH = 4096
SEQ = 200

NC = 2
NS = 16
NW = NC * NS

ROWLEN = 100
NROWS = BATCH * SEQ // ROWLEN
RW = NROWS // NW
CH = 8


def _sc_kernel(xf_hbm, tok_hbm, pos_hbm, out_hbm, idx_v, g_v, pos_v, sem):
    wid = lax.axis_index("s") * NC + lax.axis_index("c")
    r0 = wid * RW

    pltpu.sync_copy(pos_hbm, pos_v)
    pltpu.sync_copy(xf_hbm.at[pl.ds(r0, RW)], idx_v)

    def chunk_body(c, carry):
        for j in range(CH):
            pltpu.async_copy(tok_hbm.at[idx_v.at[c * CH + j]], g_v.at[j], sem)
        for j in range(CH):
            pltpu.make_async_copy(tok_hbm.at[idx_v.at[c * CH + j]],
                                  g_v.at[j], sem).wait()

        def add_body(i, acc):
            for k in range(2):
                p_lo = pos_v[i, pl.ds(k * 16, 16)]
                p_hi = pos_v[ROWLEN + i, pl.ds(k * 16, 16)]
                for j in range(CH):
                    pv = p_lo if j % 2 == 0 else p_hi
                    plsc.addupdate(g_v.at[j, i, pl.ds(k * 16, 16)], pv)
            return acc
        lax.fori_loop(0, ROWLEN, add_body, 0)

        pltpu.sync_copy(g_v, out_hbm.at[pl.ds(r0 + c * CH, CH)])
        return carry

    lax.fori_loop(0, RW // CH, chunk_body, 0)


def kernel(x, token_table, pos_table):
    xf = x.reshape(NROWS, ROWLEN).astype(jnp.int32)
    mesh = plsc.VectorSubcoreMesh(core_axis_name="c", subcore_axis_name="s",
                                  num_cores=NC, num_subcores=NS)
    out3 = pl.kernel(
        _sc_kernel,
        out_type=jax.ShapeDtypeStruct((NROWS, ROWLEN, EMBED_DIM), jnp.float32),
        mesh=mesh,
        compiler_params=pltpu.CompilerParams(use_tc_tiling_on_sc=False),
        scratch_types=[
            pltpu.VMEM((RW, ROWLEN), jnp.int32),
            pltpu.VMEM((CH, ROWLEN, EMBED_DIM), jnp.float32),
            pltpu.VMEM((MAXLEN, EMBED_DIM), jnp.float32),
            pltpu.SemaphoreType.DMA,
        ],
    )(xf, token_table, pos_table)
    return out3.reshape(BATCH, SEQ, EMBED_DIM)


# speedup vs baseline: 1.0420x; 1.0420x over previous
import jax
import jax.numpy as jnp
from jax import lax
from jax.experimental import pallas as pl
from jax.experimental.pallas import tpu as pltpu
from jax.experimental.pallas import tpu_sc as plsc

VOCAB = 1000000
MAXLEN = 200
EMBED_DIM = 32
BATCH = 4096
SEQ = 200

NC = 2
NS = 16
NW = NC * NS

ROWLEN = 100
NROWS = BATCH * SEQ // ROWLEN
RW = NROWS // NW
CH = 8


NCHUNK = RW // CH


def _sc_kernel(xf_hbm, tok_hbm, pos_hbm, out_hbm, idx_v, g_v, pos_v, *sems):
    sem_g = sems[:2]
    sem_s = sems[2:]
    wid = lax.axis_index("s") * NC + lax.axis_index("c")
    r0 = wid * RW

    pltpu.sync_copy(pos_hbm, pos_v)
    pltpu.sync_copy(xf_hbm.at[pl.ds(r0, RW)], idx_v)

    def fire_gather(c, p):
        for j in range(CH):
            pltpu.async_copy(tok_hbm.at[idx_v.at[c * CH + j]], g_v.at[p, j],
                             sem_g[p])

    def drain_gather(c, p):
        for j in range(CH):
            pltpu.make_async_copy(tok_hbm.at[idx_v.at[c * CH + j]],
                                  g_v.at[p, j], sem_g[p]).wait()

    def fire_scatter(c, p):
        pltpu.async_copy(g_v.at[p], out_hbm.at[pl.ds(r0 + c * CH, CH)],
                         sem_s[p])

    def drain_scatter(p):
        pltpu.make_async_copy(g_v.at[p], out_hbm.at[pl.ds(r0, CH)],
                              sem_s[p]).wait()

    def compute(p):
        def add_body(i, acc):
            for k in range(2):
                p_lo = pos_v[i, pl.ds(k * 16, 16)]
                p_hi = pos_v[ROWLEN + i, pl.ds(k * 16, 16)]
                for j in range(CH):
                    pv = p_lo if j % 2 == 0 else p_hi
                    plsc.addupdate(g_v.at[p, j, i, pl.ds(k * 16, 16)], pv)
            return acc
        lax.fori_loop(0, ROWLEN, add_body, 0)

    fire_gather(0, 0)

    def pair_body(c2, carry):
        for p in range(2):
            c = c2 * 2 + p
            q = 1 - p
            drain_gather(c, p)

            @pl.when(c >= 1)
            def _():
                drain_scatter(q)

            @pl.when(c + 1 < NCHUNK)
            def _():
                fire_gather(c + 1, q)
            compute(p)
            fire_scatter(c, p)
        return carry

    lax.fori_loop(0, NCHUNK // 2, pair_body, 0)
    drain_scatter((NCHUNK - 1) % 2)


def kernel(x, token_table, pos_table):
    xf = x.reshape(NROWS, ROWLEN).astype(jnp.int32)
    mesh = plsc.VectorSubcoreMesh(core_axis_name="c", subcore_axis_name="s",
                                  num_cores=NC, num_subcores=NS)
    out3 = pl.kernel(
        _sc_kernel,
        out_type=jax.ShapeDtypeStruct((NROWS, ROWLEN, EMBED_DIM), jnp.float32),
        mesh=mesh,
        compiler_params=pltpu.CompilerParams(use_tc_tiling_on_sc=False),
        scratch_types=[
            pltpu.VMEM((RW, ROWLEN), jnp.int32),
            pltpu.VMEM((2, CH, ROWLEN, EMBED_DIM), jnp.float32),
            pltpu.VMEM((MAXLEN, EMBED_DIM), jnp.float32),
        ] + [pltpu.SemaphoreType.DMA] * 4,
    )(xf, token_table, pos_table)
    return out3.reshape(BATCH, SEQ, EMBED_DIM)


# speedup vs baseline: 1.4166x; 1.3595x over previous
import jax
import jax.numpy as jnp
from jax import lax
from jax.experimental import pallas as pl
from jax.experimental.pallas import tpu as pltpu
from jax.experimental.pallas import tpu_sc as plsc

VOCAB = 1000000
MAXLEN = 200
EMBED_DIM = 32
BATCH = 4096
SEQ = 200

NC = 2
NS = 16
NW = NC * NS

BB = BATCH // NW
NBUF = 8
LOOK = 4


def _sc_kernel(xT_hbm, tok_hbm, pos_hbm, out_hbm, idx_v, g_v, pos_v, *sems):
    sem_g = sems[:NBUF]
    sem_s = sems[NBUF:]
    wid = lax.axis_index("s") * NC + lax.axis_index("c")
    b0 = wid * BB

    pltpu.sync_copy(pos_hbm, pos_v)
    pltpu.sync_copy(xT_hbm.at[:, pl.ds(b0, BB)], idx_v)

    def fire_gather(s, b):
        pltpu.async_copy(tok_hbm.at[idx_v.at[s]], g_v.at[b], sem_g[b])

    def drain_gather(s, b):
        pltpu.make_async_copy(tok_hbm.at[idx_v.at[s]], g_v.at[b],
                              sem_g[b]).wait()

    def compute(s, b):
        pv = [pos_v[s, pl.ds(k * 16, 16)] for k in range(2)]

        def r_body(rr, acc):
            for u in range(8):
                r = rr * 8 + u
                for k in range(2):
                    plsc.addupdate(g_v.at[b, r, pl.ds(k * 16, 16)], pv[k])
            return acc
        lax.fori_loop(0, BB // 8, r_body, 0)

    def fire_out(s, b):
        pltpu.async_copy(g_v.at[b], out_hbm.at[s, wid], sem_s[b])

    def drain_out(s, b):
        pltpu.make_async_copy(g_v.at[b], out_hbm.at[s, wid], sem_s[b]).wait()

    for s in range(LOOK):
        fire_gather(s, s)

    def main_body(it, carry):
        for j in range(NBUF):
            s = it * NBUF + j
            bn = (j + LOOK) % NBUF

            @pl.when(s >= NBUF - LOOK)
            def _():
                drain_out(s - (NBUF - LOOK), bn)

            @pl.when(s < SEQ - LOOK)
            def _():
                fire_gather(s + LOOK, bn)

            drain_gather(s, j)
            compute(s, j)
            fire_out(s, j)
        return carry
    lax.fori_loop(0, SEQ // NBUF, main_body, 0)

    for s in range(SEQ - (NBUF - LOOK), SEQ):
        drain_out(s, s % NBUF)


def kernel(x, token_table, pos_table):
    xT = jnp.swapaxes(x, 0, 1).astype(jnp.int32)
    mesh = plsc.VectorSubcoreMesh(core_axis_name="c", subcore_axis_name="s",
                                  num_cores=NC, num_subcores=NS)
    out4 = pl.kernel(
        _sc_kernel,
        out_type=jax.ShapeDtypeStruct((SEQ, NW, BB, EMBED_DIM), jnp.float32),
        mesh=mesh,
        compiler_params=pltpu.CompilerParams(use_tc_tiling_on_sc=False),
        scratch_types=[
            pltpu.VMEM((SEQ, BB), jnp.int32),
            pltpu.VMEM((NBUF, BB, EMBED_DIM), jnp.float32),
            pltpu.VMEM((MAXLEN, EMBED_DIM), jnp.float32),
        ] + [pltpu.SemaphoreType.DMA] * (2 * NBUF),
    )(xT, token_table, pos_table)
    return out4.reshape(SEQ, BATCH, EMBED_DIM).transpose(1, 0, 2)
